# Initial kernel scaffold; baseline (speedup 1.0000x reference)
import jax
import jax.numpy as jnp
from jax.experimental import pallas as pl


def kernel(images, mu):
    raise NotImplementedError("write your pallas kernel here")


# speedup vs baseline: 59.8670x; 59.8670x over previous
import functools

import jax
import jax.numpy as jnp
from jax.experimental import pallas as pl


def _kmeans_loss_body(x_ref, mu_ref, o_ref, *, inv_g):
    x = x_ref[...]
    m = mu_ref[...]
    dot = jnp.dot(x, m, preferred_element_type=jnp.float32)
    musq = jnp.sum(m * m, axis=0)
    xsq = jnp.sum(x * x, axis=1)
    d = musq[None, :] - 2.0 * dot
    mins = jnp.min(d, axis=1) + xsq
    o_ref[...] = (mins * inv_g)[:, None]


def kernel(images, mu):
    B, G = images.shape
    _, K = mu.shape
    out = pl.pallas_call(
        functools.partial(_kmeans_loss_body, inv_g=1.0 / G),
        out_shape=jax.ShapeDtypeStruct((B, 1), jnp.float32),
        grid=(1,),
        in_specs=[
            pl.BlockSpec((B, G), lambda i: (0, 0)),
            pl.BlockSpec((G, K), lambda i: (0, 0)),
        ],
        out_specs=pl.BlockSpec((B, 1), lambda i: (i, 0)),
    )(images, mu)
    return out[:, 0]
